# baseline (speedup 1.0000x reference)
import functools

import jax
import jax.numpy as jnp
from jax import lax
from jax.experimental import pallas as pl
from jax.experimental.pallas import tpu as pltpu
from jax.experimental.pallas import tpu_sc as plsc

HIDDEN = 768
EXPERTS = 64
TOPK = 8
N_TOK = 4 * 8192

SLICES = 1
TOK_S = N_TOK // SLICES
N_ROW_S = TOK_S // 2

MM_BLOCK = 2048

N_WORKERS = 32
ROW_PER_W = N_ROW_S // N_WORKERS
CHUNK = 64
N_CHUNK = ROW_PER_W // CHUNK


def _matmul_kernel(hs_a_ref, hs_b_ref, w_ref, logits_ref):
    dn = (((1,), (1,)), ((), ()))
    l_a = jax.lax.dot_general(hs_a_ref[...], w_ref[...], dimension_numbers=dn,
                              preferred_element_type=jnp.float32)
    l_b = jax.lax.dot_general(hs_b_ref[...], w_ref[...], dimension_numbers=dn,
                              preferred_element_type=jnp.float32)
    logits_ref[...] = jnp.concatenate([l_a, l_b], axis=1)


def _sc_topk_body(logits_hbm, scores_hbm, idx_hbm,
                  lbuf0, lbuf1, sbuf_a0, sbuf_a1, sbuf_b0, sbuf_b1,
                  ibuf_a0, ibuf_a1, ibuf_b0, ibuf_b1,
                  isem0, isem1, osem0, osem1):
    wid = lax.axis_index("s") * 2 + lax.axis_index("c")
    row_base = wid * ROW_PER_W
    lbufs = (lbuf0, lbuf1)
    sbufs_a = (sbuf_a0, sbuf_a1)
    sbufs_b = (sbuf_b0, sbuf_b1)
    ibufs_a = (ibuf_a0, ibuf_a1)
    ibufs_b = (ibuf_b0, ibuf_b1)
    isems = (isem0, isem1)
    osems = (osem0, osem1)

    lane = lax.iota(jnp.int32, 16)
    mask8 = lane < 8
    zeros16 = jnp.zeros((16,), jnp.float32)
    group_ids = [lane + 16 * g for g in range(4)]

    def merge(ka, va, kb, vb):
        mk = jnp.where(mask8, ka, jnp.flip(kb, 0))
        mv = jnp.where(mask8, va, jnp.flip(vb, 0))
        return plsc.sort_key_val(mk, mv, descending=True)

    def topk_one(lbuf, loff):
        ks, vs = [], []
        for g in range(4):
            k, v = plsc.sort_key_val(
                lbuf[pl.ds(loff + g * 16, 16)], group_ids[g], descending=True)
            ks.append(k)
            vs.append(v)
        k01, v01 = merge(ks[0], vs[0], ks[1], vs[1])
        k23, v23 = merge(ks[2], vs[2], ks[3], vs[3])
        kf, vf = merge(k01, v01, k23, v23)
        m0 = jnp.max(kf)
        e = jnp.where(mask8, jnp.exp(kf - m0), 0.0)
        s = e / jnp.sum(e)
        return s, vf

    def make_row_body(lbuf, sbuf_a, sbuf_b, ibuf_a, ibuf_b):
        def row_body(r):
            rsplat = lane * 0 + r
            s_a, v_a = topk_one(lbuf, r * 128)
            s_b, v_b = topk_one(lbuf, r * 128 + 64)
            plsc.store_scatter(sbuf_a, [rsplat, v_a], s_a, mask=mask8)
            plsc.store_scatter(sbuf_b, [rsplat, v_b], s_b, mask=mask8)
            plsc.store_scatter(ibuf_a, [rsplat, lane], v_a, mask=mask8)
            plsc.store_scatter(ibuf_b, [rsplat, lane], v_b, mask=mask8)
        return row_body

    def make_zero_body(sbuf_a, sbuf_b):
        def zero_body(i):
            r = i // 4
            c = i % 4
            sbuf_a[r, pl.ds(c * 16, 16)] = zeros16
            sbuf_b[r, pl.ds(c * 16, 16)] = zeros16
        return zero_body

    def in_copy(ci, b):
        r0 = row_base + ci * CHUNK
        return pltpu.make_async_copy(
            logits_hbm.at[pl.ds(r0 * 128, CHUNK * 128)], lbufs[b], isems[b])

    def out_copies(ci, b):
        r0 = row_base + ci * CHUNK
        return [
            pltpu.make_async_copy(
                sbufs_a[b], scores_hbm.at[pl.ds(r0, CHUNK)], osems[b]),
            pltpu.make_async_copy(
                sbufs_b[b], scores_hbm.at[pl.ds(r0 + N_ROW_S, CHUNK)],
                osems[b]),
            pltpu.make_async_copy(
                ibufs_a[b], idx_hbm.at[pl.ds(r0, CHUNK)], osems[b]),
            pltpu.make_async_copy(
                ibufs_b[b], idx_hbm.at[pl.ds(r0 + N_ROW_S, CHUNK)], osems[b]),
        ]

    in_copy(0, 0).start()
    in_copy(1, 1).start()
    for ci in range(N_CHUNK):
        b = ci % 2
        in_copy(ci, b).wait()
        plsc.parallel_loop(0, CHUNK * 4, unroll=16)(
            make_zero_body(sbufs_a[b], sbufs_b[b]))
        plsc.parallel_loop(0, CHUNK, unroll=8)(
            make_row_body(lbufs[b], sbufs_a[b], sbufs_b[b],
                          ibufs_a[b], ibufs_b[b]))
        if ci + 2 < N_CHUNK:
            in_copy(ci + 2, b).start()
        r0 = row_base + ci * CHUNK
        pltpu.sync_copy(sbufs_a[b], scores_hbm.at[pl.ds(r0, CHUNK)])
        pltpu.sync_copy(sbufs_b[b], scores_hbm.at[pl.ds(r0 + N_ROW_S, CHUNK)])
        pltpu.sync_copy(ibufs_a[b], idx_hbm.at[pl.ds(r0, CHUNK)])
        pltpu.sync_copy(ibufs_b[b], idx_hbm.at[pl.ds(r0 + N_ROW_S, CHUNK)])


@functools.cache
def _sc_topk():
    return pl.kernel(
        _sc_topk_body,
        out_type=[
            jax.ShapeDtypeStruct((TOK_S, EXPERTS), jnp.float32),
            jax.ShapeDtypeStruct((TOK_S, TOPK), jnp.int32),
        ],
        mesh=plsc.VectorSubcoreMesh(core_axis_name="c", subcore_axis_name="s",
                                    num_cores=2, num_subcores=16),
        scratch_types=[
            pltpu.VMEM((CHUNK * 128,), jnp.float32),
            pltpu.VMEM((CHUNK * 128,), jnp.float32),
            pltpu.VMEM((CHUNK, EXPERTS), jnp.float32),
            pltpu.VMEM((CHUNK, EXPERTS), jnp.float32),
            pltpu.VMEM((CHUNK, EXPERTS), jnp.float32),
            pltpu.VMEM((CHUNK, EXPERTS), jnp.float32),
            pltpu.VMEM((CHUNK, TOPK), jnp.int32),
            pltpu.VMEM((CHUNK, TOPK), jnp.int32),
            pltpu.VMEM((CHUNK, TOPK), jnp.int32),
            pltpu.VMEM((CHUNK, TOPK), jnp.int32),
            pltpu.SemaphoreType.DMA,
            pltpu.SemaphoreType.DMA,
            pltpu.SemaphoreType.DMA,
            pltpu.SemaphoreType.DMA,
        ],
        compiler_params=pltpu.CompilerParams(needs_layout_passes=False),
    )


@jax.jit
def kernel(hidden_states, weight):
    hs = hidden_states.reshape(-1, HIDDEN)
    blocks_per_half = N_ROW_S // MM_BLOCK
    scores_parts, idx_parts = [], []
    for s in range(SLICES):
        a0 = s * TOK_S // MM_BLOCK
        b0 = a0 + blocks_per_half
        logits = pl.pallas_call(
            _matmul_kernel,
            grid=(blocks_per_half,),
            in_specs=[
                pl.BlockSpec((MM_BLOCK, HIDDEN),
                             lambda i, a0=a0: (i + a0, 0)),
                pl.BlockSpec((MM_BLOCK, HIDDEN),
                             lambda i, b0=b0: (i + b0, 0)),
                pl.BlockSpec((EXPERTS, HIDDEN), lambda i: (0, 0)),
            ],
            out_specs=pl.BlockSpec((MM_BLOCK, 2 * EXPERTS), lambda i: (i, 0)),
            out_shape=jax.ShapeDtypeStruct((N_ROW_S, 2 * EXPERTS),
                                           jnp.float32),
        )(hs, hs, weight)
        sc, ix = _sc_topk()(logits.reshape(-1))
        scores_parts.append(sc)
        idx_parts.append(ix)
    if SLICES == 1:
        return scores_parts[0], idx_parts[0]
    return (jnp.concatenate(scores_parts, axis=0),
            jnp.concatenate(idx_parts, axis=0))


# speedup vs baseline: 1.0463x; 1.0463x over previous
import functools

import jax
import jax.numpy as jnp
from jax import lax
from jax.experimental import pallas as pl
from jax.experimental.pallas import tpu as pltpu
from jax.experimental.pallas import tpu_sc as plsc

HIDDEN = 768
EXPERTS = 64
TOPK = 8
N_TOK = 4 * 8192

SLICES = 1
TOK_S = N_TOK // SLICES
N_ROW_S = TOK_S // 2

MM_BLOCK = 2048

N_WORKERS = 32
ROW_PER_W = N_ROW_S // N_WORKERS
CHUNK = 128
N_CHUNK = ROW_PER_W // CHUNK


def _matmul_kernel(hs_a_ref, hs_b_ref, w_ref, logits_ref):
    dn = (((1,), (1,)), ((), ()))
    l_a = jax.lax.dot_general(hs_a_ref[...], w_ref[...], dimension_numbers=dn,
                              preferred_element_type=jnp.float32)
    l_b = jax.lax.dot_general(hs_b_ref[...], w_ref[...], dimension_numbers=dn,
                              preferred_element_type=jnp.float32)
    logits_ref[...] = jnp.concatenate([l_a, l_b], axis=1)


def _sc_topk_body(logits_hbm, scores_hbm, idx_hbm,
                  lbuf, sbuf_a, sbuf_b, ibuf_a, ibuf_b):
    wid = lax.axis_index("s") * 2 + lax.axis_index("c")
    row_base = wid * ROW_PER_W

    lane = lax.iota(jnp.int32, 16)
    mask8 = lane < 8
    zeros16 = jnp.zeros((16,), jnp.float32)
    group_ids = [lane + 16 * g for g in range(4)]

    def merge(ka, va, kb, vb):
        mk = jnp.where(mask8, ka, jnp.flip(kb, 0))
        mv = jnp.where(mask8, va, jnp.flip(vb, 0))
        return plsc.sort_key_val(mk, mv, descending=True)

    def topk_one(loff):
        ks, vs = [], []
        for g in range(4):
            k, v = plsc.sort_key_val(
                lbuf[pl.ds(loff + g * 16, 16)], group_ids[g], descending=True)
            ks.append(k)
            vs.append(v)
        k01, v01 = merge(ks[0], vs[0], ks[1], vs[1])
        k23, v23 = merge(ks[2], vs[2], ks[3], vs[3])
        kf, vf = merge(k01, v01, k23, v23)
        m0 = jnp.max(kf)
        e = jnp.where(mask8, jnp.exp(kf - m0), 0.0)
        s = e / jnp.sum(e)
        return s, vf

    def row_body(r):
        rsplat = lane * 0 + r
        s_a, v_a = topk_one(r * 128)
        s_b, v_b = topk_one(r * 128 + 64)
        plsc.store_scatter(sbuf_a, [rsplat, v_a], s_a, mask=mask8)
        plsc.store_scatter(sbuf_b, [rsplat, v_b], s_b, mask=mask8)
        plsc.store_scatter(ibuf_a, [rsplat, lane], v_a, mask=mask8)
        plsc.store_scatter(ibuf_b, [rsplat, lane], v_b, mask=mask8)

    def zero_body(i):
        r = i // 4
        c = i % 4
        sbuf_a[r, pl.ds(c * 16, 16)] = zeros16
        sbuf_b[r, pl.ds(c * 16, 16)] = zeros16

    def chunk_body(ci, _):
        r0 = row_base + ci * CHUNK
        pltpu.sync_copy(logits_hbm.at[pl.ds(r0 * 128, CHUNK * 128)], lbuf)
        plsc.parallel_loop(0, CHUNK * 4, unroll=16)(zero_body)
        plsc.parallel_loop(0, CHUNK, unroll=8)(row_body)
        pltpu.sync_copy(sbuf_a, scores_hbm.at[pl.ds(r0, CHUNK)])
        pltpu.sync_copy(sbuf_b, scores_hbm.at[pl.ds(r0 + N_ROW_S, CHUNK)])
        pltpu.sync_copy(ibuf_a, idx_hbm.at[pl.ds(r0, CHUNK)])
        pltpu.sync_copy(ibuf_b, idx_hbm.at[pl.ds(r0 + N_ROW_S, CHUNK)])
        return 0

    lax.fori_loop(0, N_CHUNK, chunk_body, 0)


@functools.cache
def _sc_topk():
    return pl.kernel(
        _sc_topk_body,
        out_type=[
            jax.ShapeDtypeStruct((TOK_S, EXPERTS), jnp.float32),
            jax.ShapeDtypeStruct((TOK_S, TOPK), jnp.int32),
        ],
        mesh=plsc.VectorSubcoreMesh(core_axis_name="c", subcore_axis_name="s",
                                    num_cores=2, num_subcores=16),
        scratch_types=[
            pltpu.VMEM((CHUNK * 128,), jnp.float32),
            pltpu.VMEM((CHUNK, EXPERTS), jnp.float32),
            pltpu.VMEM((CHUNK, EXPERTS), jnp.float32),
            pltpu.VMEM((CHUNK, TOPK), jnp.int32),
            pltpu.VMEM((CHUNK, TOPK), jnp.int32),
        ],
        compiler_params=pltpu.CompilerParams(needs_layout_passes=False),
    )


@jax.jit
def kernel(hidden_states, weight):
    hs = hidden_states.reshape(-1, HIDDEN)
    blocks_per_half = N_ROW_S // MM_BLOCK
    scores_parts, idx_parts = [], []
    for s in range(SLICES):
        a0 = s * TOK_S // MM_BLOCK
        b0 = a0 + blocks_per_half
        logits = pl.pallas_call(
            _matmul_kernel,
            grid=(blocks_per_half,),
            in_specs=[
                pl.BlockSpec((MM_BLOCK, HIDDEN),
                             lambda i, a0=a0: (i + a0, 0)),
                pl.BlockSpec((MM_BLOCK, HIDDEN),
                             lambda i, b0=b0: (i + b0, 0)),
                pl.BlockSpec((EXPERTS, HIDDEN), lambda i: (0, 0)),
            ],
            out_specs=pl.BlockSpec((MM_BLOCK, 2 * EXPERTS), lambda i: (i, 0)),
            out_shape=jax.ShapeDtypeStruct((N_ROW_S, 2 * EXPERTS),
                                           jnp.float32),
        )(hs, hs, weight)
        sc, ix = _sc_topk()(logits.reshape(-1))
        scores_parts.append(sc)
        idx_parts.append(ix)
    if SLICES == 1:
        return scores_parts[0], idx_parts[0]
    return (jnp.concatenate(scores_parts, axis=0),
            jnp.concatenate(idx_parts, axis=0))
